# baseline (speedup 1.0000x reference)
import functools

import jax
import jax.numpy as jnp
from jax import lax
from jax.experimental import pallas as pl
from jax.experimental.pallas import tpu as pltpu
from jax.experimental.pallas import tpu_sc as plsc

T, M, E = 50, 10, 4096
LANES = 16
NC, NS = 1, 16
CHUNK = E // (NC * NS)

_mesh = plsc.VectorSubcoreMesh(
    core_axis_name="c", subcore_axis_name="s", num_cores=NC, num_subcores=NS
)


@functools.partial(
    pl.kernel,
    out_type=jax.ShapeDtypeStruct((M, E), jnp.float32),
    mesh=_mesh,
    scratch_types=[
        pltpu.VMEM((LANES,), jnp.int32),
        pltpu.VMEM((M, CHUNK), jnp.float32),
        pltpu.SemaphoreType.DMA,
    ],
    compiler_params=pltpu.CompilerParams(
        use_tc_tiling_on_sc=True, needs_layout_passes=False
    ),
)
def _lookup(pool_hbm, id_hbm, out_hbm, id_v, buf, sem):
    wid = lax.axis_index("s") * NC + lax.axis_index("c")
    col = wid * CHUNK
    id_v[...] = jnp.zeros((LANES,), jnp.int32)
    pltpu.sync_copy(id_hbm, id_v.at[pl.ds(0, 1)])
    sid = jnp.max(id_v[...])
    half = CHUNK // 2
    g0 = pltpu.async_copy(
        pool_hbm.at[:, sid, pl.ds(col, half)],
        buf.at[:, pl.ds(0, half)], sem)
    g1 = pltpu.async_copy(
        pool_hbm.at[:, sid, pl.ds(col + half, half)],
        buf.at[:, pl.ds(half, half)], sem)
    g0.wait()
    s0 = pltpu.async_copy(
        buf.at[:, pl.ds(0, half)],
        out_hbm.at[:, pl.ds(col, half)], sem)
    g1.wait()
    s1 = pltpu.async_copy(
        buf.at[:, pl.ds(half, half)],
        out_hbm.at[:, pl.ds(col + half, half)], sem)
    s0.wait()
    s1.wait()


def kernel(pool, id):
    pool_t = jnp.transpose(pool, (1, 0, 2))
    id_vec = jnp.reshape(id, (1,)).astype(jnp.int32)
    return _lookup(pool_t, id_vec)
